# Initial kernel scaffold; baseline (speedup 1.0000x reference)
import jax
import jax.numpy as jnp
from jax.experimental import pallas as pl


def kernel(inputs, W1, b1, W2, b2, edge_table, source_indices, dest_indices, edge_types):
    raise NotImplementedError("write your pallas kernel here")


# speedup vs baseline: 19.4025x; 19.4025x over previous
import functools

import jax
import jax.numpy as jnp
from jax import lax
from jax.experimental import pallas as pl
from jax.experimental.pallas import tpu as pltpu
from jax.experimental.pallas import tpu_sc as plsc

N = 10000
E = 320000
H = 128
T = 6
TP = 8
NC = 2
NS = 16
NW = NC * NS
EW = E // NW
K = 80
NB = EW // K
NP = 10240
RPS = NP // NS
ZR = 128
B2W = NP * TP
B2S = B2W // NS
RB = 1000


def _tc_head_body(x_ref, w1_ref, b1_ref, w2_ref, keys_ref, ab_ref):
    k = jnp.dot(x_ref[...], w1_ref[...], preferred_element_type=jnp.float32)
    k = k + b1_ref[...]
    keys_ref[...] = k
    ab_ref[...] = jnp.dot(k, w2_ref[...], preferred_element_type=jnp.float32)


def _tc_head(x, w1, b1r, w2m):
    return pl.pallas_call(
        _tc_head_body,
        grid=(N // RB,),
        in_specs=[
            pl.BlockSpec((RB, H), lambda i: (i, 0)),
            pl.BlockSpec((H, H), lambda i: (0, 0)),
            pl.BlockSpec((1, H), lambda i: (0, 0)),
            pl.BlockSpec((H, 2), lambda i: (0, 0)),
        ],
        out_specs=[
            pl.BlockSpec((RB, H), lambda i: (i, 0)),
            pl.BlockSpec((RB, 2), lambda i: (i, 0)),
        ],
        out_shape=[
            jax.ShapeDtypeStruct((N, H), jnp.float32),
            jax.ShapeDtypeStruct((N, 2), jnp.float32),
        ],
    )(x, w1, b1r, w2m)


def _sc_body(keys_hbm, ab_hbm, c6_hbm, src_hbm, dst_hbm, et_hbm,
             z2d_hbm, z1d_hbm, outp_hbm, b2p_hbm,
             ab_vm, c6_vm, sidx, didx, etb, fidx, pbuf, rows,
             out_sh, b2_sh, sem):
    cid = lax.axis_index("c")
    sid = lax.axis_index("s")
    wid = sid * NC + cid

    pltpu.sync_copy(ab_hbm, ab_vm)
    pltpu.sync_copy(c6_hbm, c6_vm)

    for z in range(RPS // ZR):
        pltpu.sync_copy(z2d_hbm, out_sh.at[pl.ds(sid * RPS + z * ZR, ZR)])
    pltpu.sync_copy(z1d_hbm, b2_sh.at[pl.ds(sid * B2S, B2S)])
    plsc.subcore_barrier()

    one16 = jnp.ones((16,), jnp.int32)
    ebase = wid * EW

    @pl.loop(0, NB)
    def _block(j):
        base = ebase + j * K
        pltpu.sync_copy(src_hbm.at[pl.ds(base, K)], sidx)
        pltpu.sync_copy(dst_hbm.at[pl.ds(base, K)], didx)
        pltpu.sync_copy(et_hbm.at[pl.ds(base, K)], etb)
        gat = pltpu.async_copy(keys_hbm.at[sidx], rows, sem)
        for i in range(K // 16):
            sl = pl.ds(i * 16, 16)
            s16 = sidx[sl]
            d16 = didx[sl]
            t16 = etb[sl]
            av = plsc.load_gather(ab_vm, [s16 * 2])
            bv = plsc.load_gather(ab_vm, [d16 * 2 + one16])
            cv = plsc.load_gather(c6_vm, [t16])
            logit = av + bv + cv
            logit = jnp.maximum(logit, 0.2 * logit)
            pbuf[sl] = jnp.exp(logit)
            fidx[sl] = d16 * TP + t16
        gat.wait()

        @pl.loop(0, K // 16)
        def _scale(g):
            p16 = pbuf[pl.ds(g * 16, 16)]
            for l in range(16):
                ps = p16[l]
                i = g * 16 + l
                for h in range(H // 16):
                    hsl = pl.ds(h * 16, 16)
                    rows[i, hsl] = rows[i, hsl] * ps

        pltpu.sync_copy(pbuf, b2_sh.at[fidx], add=True)
        pltpu.sync_copy(rows, out_sh.at[didx], add=True)

    plsc.subcore_barrier()

    for z in range(RPS // ZR):
        r0 = sid * RPS + z * ZR
        pltpu.sync_copy(out_sh.at[pl.ds(r0, ZR)], outp_hbm.at[cid, pl.ds(r0, ZR)])
    pltpu.sync_copy(b2_sh.at[pl.ds(sid * B2S, B2S)],
                    b2p_hbm.at[cid, pl.ds(sid * B2S, B2S)])


_sc_edges = functools.partial(
    pl.kernel,
    out_type=(
        jax.ShapeDtypeStruct((NC, NP, H), jnp.float32),
        jax.ShapeDtypeStruct((NC, B2W), jnp.float32),
    ),
    mesh=plsc.VectorSubcoreMesh(
        core_axis_name="c", subcore_axis_name="s", num_cores=NC,
        num_subcores=NS),
    compiler_params=pltpu.CompilerParams(needs_layout_passes=False),
    scratch_types=[
        pltpu.VMEM((2 * N,), jnp.float32),
        pltpu.VMEM((16,), jnp.float32),
        pltpu.VMEM((K,), jnp.int32),
        pltpu.VMEM((K,), jnp.int32),
        pltpu.VMEM((K,), jnp.int32),
        pltpu.VMEM((K,), jnp.int32),
        pltpu.VMEM((K,), jnp.float32),
        pltpu.VMEM((K, H), jnp.float32),
        pltpu.VMEM_SHARED((NP, H), jnp.float32),
        pltpu.VMEM_SHARED((B2W,), jnp.float32),
        pltpu.SemaphoreType.DMA,
    ],
)(_sc_body)


def _tc_norm_body(p_ref, b2_ref, et_ref, o_ref):
    b2 = b2_ref[0] + b2_ref[1]
    s = jnp.sum(b2, axis=-1, keepdims=True)
    et2 = jnp.dot(b2, et_ref[...], preferred_element_type=jnp.float32)
    num = p_ref[0] + p_ref[1] + et2
    o_ref[...] = num * jnp.where(s > 0.0, 1.0 / s, 0.0)


def _tc_norm(p, b2r, etp):
    return pl.pallas_call(
        _tc_norm_body,
        grid=(N // RB,),
        in_specs=[
            pl.BlockSpec((NC, RB, H), lambda i: (0, i, 0)),
            pl.BlockSpec((NC, RB, TP), lambda i: (0, i, 0)),
            pl.BlockSpec((TP, H), lambda i: (0, 0)),
        ],
        out_specs=pl.BlockSpec((RB, H), lambda i: (i, 0)),
        out_shape=jax.ShapeDtypeStruct((N, H), jnp.float32),
    )(p, b2r, etp)


def kernel(inputs, W1, b1, W2, b2, edge_table, source_indices, dest_indices,
           edge_types):
    w2m = jnp.concatenate([W2[:H], W2[H:]], axis=1)
    keys, ab = _tc_head(inputs, W1, b1.reshape(1, H), w2m)
    c6 = jnp.pad(edge_table @ W2[:H, 0] + b2[0], (0, 16 - T))
    z2d = jnp.zeros((ZR, H), jnp.float32)
    z1d = jnp.zeros((B2S,), jnp.float32)
    outp, b2p = _sc_edges(
        keys, ab.reshape(2 * N), c6,
        source_indices.astype(jnp.int32),
        dest_indices.astype(jnp.int32),
        edge_types.astype(jnp.int32),
        z2d, z1d)
    etp = jnp.pad(edge_table, ((0, TP - T), (0, 0)))
    out = _tc_norm(outp, b2p.reshape(NC, NP, TP), etp)
    return out


# speedup vs baseline: 28.9376x; 1.4914x over previous
import functools

import jax
import jax.numpy as jnp
from jax import lax
from jax.experimental import pallas as pl
from jax.experimental.pallas import tpu as pltpu
from jax.experimental.pallas import tpu_sc as plsc

N = 10000
E = 320000
H = 128
T = 6
TP = 8
NC = 2
NS = 16
NW = NC * NS
EW = E // NW
K = 80
NB = EW // K
NP = 10240
RPS = NP // NS
ZR = 128
B2W = NP * TP
B2S = B2W // NS
RB = 1000


def _tc_head_body(x_ref, w1_ref, b1_ref, w2_ref, keys_ref, ab_ref):
    k = jnp.dot(x_ref[...], w1_ref[...], preferred_element_type=jnp.float32)
    k = k + b1_ref[...]
    keys_ref[...] = k
    ab_ref[...] = jnp.dot(k, w2_ref[...], preferred_element_type=jnp.float32)


def _tc_head(x, w1, b1r, w2m):
    return pl.pallas_call(
        _tc_head_body,
        grid=(N // RB,),
        in_specs=[
            pl.BlockSpec((RB, H), lambda i: (i, 0)),
            pl.BlockSpec((H, H), lambda i: (0, 0)),
            pl.BlockSpec((1, H), lambda i: (0, 0)),
            pl.BlockSpec((H, 2), lambda i: (0, 0)),
        ],
        out_specs=[
            pl.BlockSpec((RB, H), lambda i: (i, 0)),
            pl.BlockSpec((RB, 2), lambda i: (i, 0)),
        ],
        out_shape=[
            jax.ShapeDtypeStruct((N, H), jnp.float32),
            jax.ShapeDtypeStruct((N, 2), jnp.float32),
        ],
    )(x, w1, b1r, w2m)


def _sc_body(keys_hbm, ab_hbm, c6_hbm, src_hbm, dst_hbm, et_hbm,
             z2d_hbm, z1d_hbm, outp_hbm, b2p_hbm,
             ab_vm, c6_vm,
             sidx0, didx0, etb0, fidx0, pbuf0, rows0,
             sidx1, didx1, etb1, fidx1, pbuf1, rows1,
             out_sh, b2_sh,
             sem_i0, sem_i1, sem_g0, sem_g1, sem_s0, sem_s1):
    cid = lax.axis_index("c")
    sid = lax.axis_index("s")
    wid = sid * NC + cid

    pltpu.sync_copy(ab_hbm, ab_vm)
    pltpu.sync_copy(c6_hbm, c6_vm)

    for z in range(RPS // ZR):
        pltpu.sync_copy(z2d_hbm, out_sh.at[pl.ds(sid * RPS + z * ZR, ZR)])
    pltpu.sync_copy(z1d_hbm, b2_sh.at[pl.ds(sid * B2S, B2S)])
    plsc.subcore_barrier()

    one16 = jnp.ones((16,), jnp.int32)
    ebase = wid * EW
    BUF = ((sidx0, didx0, etb0, fidx0, pbuf0, rows0, sem_i0, sem_g0, sem_s0),
           (sidx1, didx1, etb1, fidx1, pbuf1, rows1, sem_i1, sem_g1, sem_s1))

    def issue_idx(bi, j):
        sidx, didx, etb, _, _, _, si, _, _ = BUF[bi]
        base = ebase + j * K
        pltpu.async_copy(src_hbm.at[pl.ds(base, K)], sidx, si)
        pltpu.async_copy(dst_hbm.at[pl.ds(base, K)], didx, si)
        pltpu.async_copy(et_hbm.at[pl.ds(base, K)], etb, si)

    def drain_idx(bi):
        sidx, didx, etb, _, _, _, si, _, _ = BUF[bi]
        pltpu.make_async_copy(src_hbm.at[pl.ds(0, K)], sidx, si).wait()
        pltpu.make_async_copy(dst_hbm.at[pl.ds(0, K)], didx, si).wait()
        pltpu.make_async_copy(et_hbm.at[pl.ds(0, K)], etb, si).wait()

    def drain_scat(bi):
        _, didx, _, fidx, pbuf, rows, _, _, ss = BUF[bi]
        pltpu.make_async_copy(rows, out_sh.at[didx], ss).wait()
        pltpu.make_async_copy(pbuf, b2_sh.at[fidx], ss).wait()

    def step(j, bi, drain_prev=True, prefetch=True, mguard=None):
        sidx, didx, etb, fidx, pbuf, rows, _, sg, ss = BUF[bi]
        if mguard is not None:
            @pl.when(mguard)
            def _():
                drain_scat(1 - bi)
        elif drain_prev:
            drain_scat(1 - bi)
        if prefetch:
            issue_idx(1 - bi, j + 1)
        drain_idx(bi)
        gat = pltpu.async_copy(keys_hbm.at[sidx], rows, sg)
        for i in range(K // 16):
            sl = pl.ds(i * 16, 16)
            s16 = sidx[sl]
            d16 = didx[sl]
            t16 = etb[sl]
            av = plsc.load_gather(ab_vm, [s16 * 2])
            bv = plsc.load_gather(ab_vm, [d16 * 2 + one16])
            cv = plsc.load_gather(c6_vm, [t16])
            logit = av + bv + cv
            logit = jnp.maximum(logit, 0.2 * logit)
            pbuf[sl] = jnp.exp(logit)
            fidx[sl] = d16 * TP + t16
        gat.wait()

        @pl.loop(0, K // 16)
        def _scale(g):
            p16 = pbuf[pl.ds(g * 16, 16)]
            for l in range(16):
                ps = p16[l]
                i = g * 16 + l
                for h in range(H // 16):
                    hsl = pl.ds(h * 16, 16)
                    rows[i, hsl] = rows[i, hsl] * ps

        pltpu.async_copy(rows, out_sh.at[didx], ss, add=True)
        pltpu.async_copy(pbuf, b2_sh.at[fidx], ss, add=True)

    issue_idx(0, 0)

    @pl.loop(0, (NB - 1) // 2)
    def _pair(m):
        j0 = 2 * m
        step(j0, 0, mguard=m > 0)
        step(j0 + 1, 1)

    step(NB - 1, 0, prefetch=False)
    drain_scat(0)

    plsc.subcore_barrier()

    for z in range(RPS // ZR):
        r0 = sid * RPS + z * ZR
        pltpu.sync_copy(out_sh.at[pl.ds(r0, ZR)], outp_hbm.at[cid, pl.ds(r0, ZR)])
    pltpu.sync_copy(b2_sh.at[pl.ds(sid * B2S, B2S)],
                    b2p_hbm.at[cid, pl.ds(sid * B2S, B2S)])


_sc_edges = functools.partial(
    pl.kernel,
    out_type=(
        jax.ShapeDtypeStruct((NC, NP, H), jnp.float32),
        jax.ShapeDtypeStruct((NC, B2W), jnp.float32),
    ),
    mesh=plsc.VectorSubcoreMesh(
        core_axis_name="c", subcore_axis_name="s", num_cores=NC,
        num_subcores=NS),
    compiler_params=pltpu.CompilerParams(needs_layout_passes=False),
    scratch_types=[
        pltpu.VMEM((2 * N,), jnp.float32),
        pltpu.VMEM((16,), jnp.float32),
        pltpu.VMEM((K,), jnp.int32),
        pltpu.VMEM((K,), jnp.int32),
        pltpu.VMEM((K,), jnp.int32),
        pltpu.VMEM((K,), jnp.int32),
        pltpu.VMEM((K,), jnp.float32),
        pltpu.VMEM((K, H), jnp.float32),
        pltpu.VMEM((K,), jnp.int32),
        pltpu.VMEM((K,), jnp.int32),
        pltpu.VMEM((K,), jnp.int32),
        pltpu.VMEM((K,), jnp.int32),
        pltpu.VMEM((K,), jnp.float32),
        pltpu.VMEM((K, H), jnp.float32),
        pltpu.VMEM_SHARED((NP, H), jnp.float32),
        pltpu.VMEM_SHARED((B2W,), jnp.float32),
        pltpu.SemaphoreType.DMA,
        pltpu.SemaphoreType.DMA,
        pltpu.SemaphoreType.DMA,
        pltpu.SemaphoreType.DMA,
        pltpu.SemaphoreType.DMA,
        pltpu.SemaphoreType.DMA,
    ],
)(_sc_body)


def _tc_norm_body(p_ref, b2_ref, et_ref, o_ref):
    b2 = b2_ref[0] + b2_ref[1]
    s = jnp.sum(b2, axis=-1, keepdims=True)
    et2 = jnp.dot(b2, et_ref[...], preferred_element_type=jnp.float32)
    num = p_ref[0] + p_ref[1] + et2
    o_ref[...] = num * jnp.where(s > 0.0, 1.0 / s, 0.0)


def _tc_norm(p, b2r, etp):
    return pl.pallas_call(
        _tc_norm_body,
        grid=(N // RB,),
        in_specs=[
            pl.BlockSpec((NC, RB, H), lambda i: (0, i, 0)),
            pl.BlockSpec((NC, RB, TP), lambda i: (0, i, 0)),
            pl.BlockSpec((TP, H), lambda i: (0, 0)),
        ],
        out_specs=pl.BlockSpec((RB, H), lambda i: (i, 0)),
        out_shape=jax.ShapeDtypeStruct((N, H), jnp.float32),
    )(p, b2r, etp)


def kernel(inputs, W1, b1, W2, b2, edge_table, source_indices, dest_indices,
           edge_types):
    w2m = jnp.concatenate([W2[:H], W2[H:]], axis=1)
    keys, ab = _tc_head(inputs, W1, b1.reshape(1, H), w2m)
    c6 = jnp.pad(edge_table @ W2[:H, 0] + b2[0], (0, 16 - T))
    z2d = jnp.zeros((ZR, H), jnp.float32)
    z1d = jnp.zeros((B2S,), jnp.float32)
    outp, b2p = _sc_edges(
        keys, ab.reshape(2 * N), c6,
        source_indices.astype(jnp.int32),
        dest_indices.astype(jnp.int32),
        edge_types.astype(jnp.int32),
        z2d, z1d)
    etp = jnp.pad(edge_table, ((0, TP - T), (0, 0)))
    out = _tc_norm(outp, b2p.reshape(NC, NP, TP), etp)
    return out


# speedup vs baseline: 41.6905x; 1.4407x over previous
import functools

import jax
import jax.numpy as jnp
from jax import lax
from jax.experimental import pallas as pl
from jax.experimental.pallas import tpu as pltpu
from jax.experimental.pallas import tpu_sc as plsc

N = 10000
E = 320000
H = 128
T = 6
TP = 8
NC = 2
NS = 16
NW = NC * NS
EW = E // NW
K = 80
NB = EW // K
NP = 10240
RPS = NP // NS
ZR = 128
B2W = NP * TP
B2S = B2W // NS
RB = 1000


def _tc_head_body(x_ref, w1_ref, b1_ref, w2_ref, keys_ref, ab_ref):
    k = jnp.dot(x_ref[...], w1_ref[...], preferred_element_type=jnp.float32)
    k = k + b1_ref[...]
    keys_ref[...] = k
    ab_ref[...] = jnp.dot(k, w2_ref[...], preferred_element_type=jnp.float32)


def _tc_head(x, w1, b1r, w2m):
    return pl.pallas_call(
        _tc_head_body,
        grid=(N // RB,),
        in_specs=[
            pl.BlockSpec((RB, H), lambda i: (i, 0)),
            pl.BlockSpec((H, H), lambda i: (0, 0)),
            pl.BlockSpec((1, H), lambda i: (0, 0)),
            pl.BlockSpec((H, 2), lambda i: (0, 0)),
        ],
        out_specs=[
            pl.BlockSpec((RB, H), lambda i: (i, 0)),
            pl.BlockSpec((RB, 2), lambda i: (i, 0)),
        ],
        out_shape=[
            jax.ShapeDtypeStruct((N, H), jnp.float32),
            jax.ShapeDtypeStruct((N, 2), jnp.float32),
        ],
    )(x, w1, b1r, w2m)


def _sc_body(keys_hbm, ab_hbm, c6_hbm, src_hbm, dst_hbm, et_hbm,
             z2d_hbm, z1d_hbm, outp_hbm, b2p_hbm,
             ab_vm, c6_vm,
             sidx0, didx0, etb0, fidx0, pbuf0, rows0, sdid0,
             sidx1, didx1, etb1, fidx1, pbuf1, rows1, sdid1,
             out_sh, b2_sh,
             sem_i0, sem_i1, sem_g0, sem_g1, sem_s0, sem_s1):
    cid = lax.axis_index("c")
    sid = lax.axis_index("s")
    wid = sid * NC + cid

    pltpu.sync_copy(ab_hbm, ab_vm)
    pltpu.sync_copy(c6_hbm, c6_vm)

    for z in range(RPS // ZR):
        pltpu.sync_copy(z2d_hbm, out_sh.at[pl.ds(sid * RPS + z * ZR, ZR)])
    pltpu.sync_copy(z1d_hbm, b2_sh.at[pl.ds(sid * B2S, B2S)])
    plsc.subcore_barrier()

    one16 = jnp.ones((16,), jnp.int32)
    ebase = wid * EW
    IX = ((sidx0, didx0, etb0, sem_i0), (sidx1, didx1, etb1, sem_i1))
    SC = ((fidx0, pbuf0, rows0, sdid0, sem_g0, sem_s0),
          (fidx1, pbuf1, rows1, sdid1, sem_g1, sem_s1))

    def issue_idx(p, j):
        sidx, didx, etb, si = IX[p]
        base = ebase + j * K
        pltpu.async_copy(src_hbm.at[pl.ds(base, K)], sidx, si)
        pltpu.async_copy(dst_hbm.at[pl.ds(base, K)], didx, si)
        pltpu.async_copy(et_hbm.at[pl.ds(base, K)], etb, si)

    def drain_idx(p):
        sidx, didx, etb, si = IX[p]
        pltpu.make_async_copy(src_hbm.at[pl.ds(0, K)], sidx, si).wait()
        pltpu.make_async_copy(dst_hbm.at[pl.ds(0, K)], didx, si).wait()
        pltpu.make_async_copy(et_hbm.at[pl.ds(0, K)], etb, si).wait()

    def drain_scat(p):
        fidx, pbuf, rows, sdid, _, ss = SC[p]
        pltpu.make_async_copy(rows, out_sh.at[sdid], ss).wait()
        pltpu.make_async_copy(pbuf, b2_sh.at[fidx], ss).wait()

    def issue_gather(p):
        sidx, _, _, _ = IX[p]
        _, _, rows, _, sg, _ = SC[p]
        pltpu.async_copy(keys_hbm.at[sidx], rows, sg)

    def wait_gather(p):
        sidx, _, _, _ = IX[p]
        _, _, rows, _, sg, _ = SC[p]
        pltpu.make_async_copy(keys_hbm.at[sidx], rows, sg).wait()

    def compute_p(p):
        sidx, didx, etb, _ = IX[p]
        fidx, pbuf, _, sdid, _, _ = SC[p]
        for i in range(K // 16):
            sl = pl.ds(i * 16, 16)
            s16 = sidx[sl]
            d16 = didx[sl]
            t16 = etb[sl]
            av = plsc.load_gather(ab_vm, [s16 * 2])
            bv = plsc.load_gather(ab_vm, [d16 * 2 + one16])
            cv = plsc.load_gather(c6_vm, [t16])
            logit = av + bv + cv
            logit = jnp.maximum(logit, 0.2 * logit)
            pbuf[sl] = jnp.exp(logit)
            fidx[sl] = d16 * TP + t16
            sdid[sl] = d16

    def scale_and_scat(p):
        fidx, pbuf, rows, sdid, _, ss = SC[p]

        @pl.loop(0, K // 16)
        def _scale(g):
            p16 = pbuf[pl.ds(g * 16, 16)]
            for l in range(16):
                ps = p16[l]
                i = g * 16 + l
                for h in range(H // 16):
                    hsl = pl.ds(h * 16, 16)
                    rows[i, hsl] = rows[i, hsl] * ps

        pltpu.async_copy(rows, out_sh.at[sdid], ss, add=True)
        pltpu.async_copy(pbuf, b2_sh.at[fidx], ss, add=True)

    def step(j, p, drain2=True, prefetch=True, scale_prev=True):
        if drain2:
            drain_scat(p)
        drain_idx(p)
        issue_gather(p)
        compute_p(p)
        if prefetch:
            issue_idx(1 - p, j + 1)
        if scale_prev:
            scale_and_scat(1 - p)
        wait_gather(p)

    issue_idx(0, 0)
    step(0, 0, drain2=False, scale_prev=False)
    step(1, 1, drain2=False)

    @pl.loop(0, (NB - 3) // 2)
    def _pair(m):
        j0 = 2 * m + 2
        step(j0, 0)
        step(j0 + 1, 1)

    step(NB - 1, 0, prefetch=False)
    scale_and_scat(0)
    drain_scat(1)
    drain_scat(0)

    plsc.subcore_barrier()

    for z in range(RPS // ZR):
        r0 = sid * RPS + z * ZR
        pltpu.sync_copy(out_sh.at[pl.ds(r0, ZR)], outp_hbm.at[cid, pl.ds(r0, ZR)])
    pltpu.sync_copy(b2_sh.at[pl.ds(sid * B2S, B2S)],
                    b2p_hbm.at[cid, pl.ds(sid * B2S, B2S)])


_sc_edges = functools.partial(
    pl.kernel,
    out_type=(
        jax.ShapeDtypeStruct((NC, NP, H), jnp.float32),
        jax.ShapeDtypeStruct((NC, B2W), jnp.float32),
    ),
    mesh=plsc.VectorSubcoreMesh(
        core_axis_name="c", subcore_axis_name="s", num_cores=NC,
        num_subcores=NS),
    compiler_params=pltpu.CompilerParams(needs_layout_passes=False),
    scratch_types=[
        pltpu.VMEM((2 * N,), jnp.float32),
        pltpu.VMEM((16,), jnp.float32),
        pltpu.VMEM((K,), jnp.int32),
        pltpu.VMEM((K,), jnp.int32),
        pltpu.VMEM((K,), jnp.int32),
        pltpu.VMEM((K,), jnp.int32),
        pltpu.VMEM((K,), jnp.float32),
        pltpu.VMEM((K, H), jnp.float32),
        pltpu.VMEM((K,), jnp.int32),
        pltpu.VMEM((K,), jnp.int32),
        pltpu.VMEM((K,), jnp.int32),
        pltpu.VMEM((K,), jnp.int32),
        pltpu.VMEM((K,), jnp.int32),
        pltpu.VMEM((K,), jnp.float32),
        pltpu.VMEM((K, H), jnp.float32),
        pltpu.VMEM((K,), jnp.int32),
        pltpu.VMEM_SHARED((NP, H), jnp.float32),
        pltpu.VMEM_SHARED((B2W,), jnp.float32),
        pltpu.SemaphoreType.DMA,
        pltpu.SemaphoreType.DMA,
        pltpu.SemaphoreType.DMA,
        pltpu.SemaphoreType.DMA,
        pltpu.SemaphoreType.DMA,
        pltpu.SemaphoreType.DMA,
    ],
)(_sc_body)


def _tc_norm_body(p_ref, b2_ref, et_ref, o_ref):
    b2 = b2_ref[0] + b2_ref[1]
    s = jnp.sum(b2, axis=-1, keepdims=True)
    et2 = jnp.dot(b2, et_ref[...], preferred_element_type=jnp.float32)
    num = p_ref[0] + p_ref[1] + et2
    o_ref[...] = num * jnp.where(s > 0.0, 1.0 / s, 0.0)


def _tc_norm(p, b2r, etp):
    return pl.pallas_call(
        _tc_norm_body,
        grid=(N // RB,),
        in_specs=[
            pl.BlockSpec((NC, RB, H), lambda i: (0, i, 0)),
            pl.BlockSpec((NC, RB, TP), lambda i: (0, i, 0)),
            pl.BlockSpec((TP, H), lambda i: (0, 0)),
        ],
        out_specs=pl.BlockSpec((RB, H), lambda i: (i, 0)),
        out_shape=jax.ShapeDtypeStruct((N, H), jnp.float32),
    )(p, b2r, etp)


def kernel(inputs, W1, b1, W2, b2, edge_table, source_indices, dest_indices,
           edge_types):
    w2m = jnp.concatenate([W2[:H], W2[H:]], axis=1)
    keys, ab = _tc_head(inputs, W1, b1.reshape(1, H), w2m)
    c6 = jnp.pad(edge_table @ W2[:H, 0] + b2[0], (0, 16 - T))
    z2d = jnp.zeros((ZR, H), jnp.float32)
    z1d = jnp.zeros((B2S,), jnp.float32)
    outp, b2p = _sc_edges(
        keys, ab.reshape(2 * N), c6,
        source_indices.astype(jnp.int32),
        dest_indices.astype(jnp.int32),
        edge_types.astype(jnp.int32),
        z2d, z1d)
    etp = jnp.pad(edge_table, ((0, TP - T), (0, 0)))
    out = _tc_norm(outp, b2p.reshape(NC, NP, TP), etp)
    return out


# speedup vs baseline: 41.6973x; 1.0002x over previous
import functools

import jax
import jax.numpy as jnp
from jax import lax
from jax.experimental import pallas as pl
from jax.experimental.pallas import tpu as pltpu
from jax.experimental.pallas import tpu_sc as plsc

N = 10000
E = 320000
H = 128
T = 6
TP = 8
NC = 2
NS = 16
NW = NC * NS
EW = E // NW
K = 80
NB = EW // K
NP = 10240
RPS = NP // NS
ZR = 128
B2W = NP * TP
B2S = B2W // NS
RB = 1000


def _tc_head_body(x_ref, w1_ref, b1_ref, w2_ref, keys_ref, ab_ref):
    k = jnp.dot(x_ref[...], w1_ref[...], preferred_element_type=jnp.float32)
    k = k + b1_ref[...]
    keys_ref[...] = k
    ab_ref[...] = jnp.dot(k, w2_ref[...], preferred_element_type=jnp.float32)


def _tc_head(x, w1, b1r, w2m):
    return pl.pallas_call(
        _tc_head_body,
        grid=(N // RB,),
        in_specs=[
            pl.BlockSpec((RB, H), lambda i: (i, 0)),
            pl.BlockSpec((H, H), lambda i: (0, 0)),
            pl.BlockSpec((1, H), lambda i: (0, 0)),
            pl.BlockSpec((H, 2), lambda i: (0, 0)),
        ],
        out_specs=[
            pl.BlockSpec((RB, H), lambda i: (i, 0)),
            pl.BlockSpec((RB, 2), lambda i: (i, 0)),
        ],
        out_shape=[
            jax.ShapeDtypeStruct((N, H), jnp.float32),
            jax.ShapeDtypeStruct((N, 2), jnp.float32),
        ],
    )(x, w1, b1r, w2m)


def _sc_body(keys_hbm, ab_hbm, c6_hbm, src_hbm, dst_hbm, et_hbm,
             z2d_hbm, z1d_hbm, outp_hbm, b2p_hbm,
             ab_vm, c6_vm,
             sidx0, didx0, etb0, fidx0, pbuf0, rows0, sdid0,
             sidx1, didx1, etb1, fidx1, pbuf1, rows1, sdid1,
             out_sh, b2_sh,
             sem_i0, sem_i1, sem_g0, sem_g1, sem_s0, sem_s1):
    cid = lax.axis_index("c")
    sid = lax.axis_index("s")
    wid = sid * NC + cid

    pltpu.sync_copy(ab_hbm, ab_vm)
    pltpu.sync_copy(c6_hbm, c6_vm)

    for z in range(RPS // ZR):
        pltpu.sync_copy(z2d_hbm, out_sh.at[pl.ds(sid * RPS + z * ZR, ZR)])
    pltpu.sync_copy(z1d_hbm, b2_sh.at[pl.ds(sid * B2S, B2S)])
    plsc.subcore_barrier()

    one16 = jnp.ones((16,), jnp.int32)
    ebase = wid * EW
    IX = ((sidx0, didx0, etb0, sem_i0), (sidx1, didx1, etb1, sem_i1))
    SC = ((fidx0, pbuf0, rows0, sdid0, sem_g0, sem_s0),
          (fidx1, pbuf1, rows1, sdid1, sem_g1, sem_s1))

    def issue_idx(p, j):
        sidx, didx, etb, si = IX[p]
        base = ebase + j * K
        pltpu.async_copy(src_hbm.at[pl.ds(base, K)], sidx, si)
        pltpu.async_copy(dst_hbm.at[pl.ds(base, K)], didx, si)
        pltpu.async_copy(et_hbm.at[pl.ds(base, K)], etb, si)

    def drain_idx(p):
        sidx, didx, etb, si = IX[p]
        pltpu.make_async_copy(src_hbm.at[pl.ds(0, K)], sidx, si).wait()
        pltpu.make_async_copy(dst_hbm.at[pl.ds(0, K)], didx, si).wait()
        pltpu.make_async_copy(et_hbm.at[pl.ds(0, K)], etb, si).wait()

    def drain_scat(p):
        fidx, pbuf, rows, sdid, _, ss = SC[p]
        pltpu.make_async_copy(rows, out_sh.at[sdid], ss).wait()
        pltpu.make_async_copy(pbuf, b2_sh.at[fidx], ss).wait()

    def issue_gather(p):
        sidx, _, _, _ = IX[p]
        _, _, rows, _, sg, _ = SC[p]
        pltpu.async_copy(keys_hbm.at[sidx], rows, sg)

    def wait_gather(p):
        sidx, _, _, _ = IX[p]
        _, _, rows, _, sg, _ = SC[p]
        pltpu.make_async_copy(keys_hbm.at[sidx], rows, sg).wait()

    def compute_p(p):
        sidx, didx, etb, _ = IX[p]
        fidx, pbuf, _, sdid, _, _ = SC[p]
        for i in range(K // 16):
            sl = pl.ds(i * 16, 16)
            s16 = sidx[sl]
            d16 = didx[sl]
            t16 = etb[sl]
            av = plsc.load_gather(ab_vm, [s16 * 2])
            bv = plsc.load_gather(ab_vm, [d16 * 2 + one16])
            cv = plsc.load_gather(c6_vm, [t16])
            logit = av + bv + cv
            logit = jnp.maximum(logit, 0.2 * logit)
            pbuf[sl] = jnp.exp(logit)
            fidx[sl] = d16 * TP + t16
            sdid[sl] = d16

    def scale_and_scat(p):
        fidx, pbuf, rows, sdid, _, ss = SC[p]

        @pl.loop(0, K // 16)
        def _scale(g):
            p16 = pbuf[pl.ds(g * 16, 16)]
            for l in range(16):
                ps = p16[l]
                i = g * 16 + l
                for h in range(H // 16):
                    hsl = pl.ds(h * 16, 16)
                    rows[i, hsl] = rows[i, hsl] * ps

        pltpu.async_copy(rows, out_sh.at[sdid], ss, add=True)
        pltpu.async_copy(pbuf, b2_sh.at[fidx], ss, add=True)

    def step(j, p, drain2=True, prefetch=True, scale_prev=True):
        if prefetch:
            issue_idx(1 - p, j + 1)
        if drain2:
            drain_scat(p)
        drain_idx(p)
        issue_gather(p)
        compute_p(p)
        if scale_prev:
            scale_and_scat(1 - p)
        wait_gather(p)

    issue_idx(0, 0)
    step(0, 0, drain2=False, scale_prev=False)
    step(1, 1, drain2=False)

    @pl.loop(0, (NB - 3) // 2)
    def _pair(m):
        j0 = 2 * m + 2
        step(j0, 0)
        step(j0 + 1, 1)

    step(NB - 1, 0, prefetch=False)
    scale_and_scat(0)
    drain_scat(1)
    drain_scat(0)

    plsc.subcore_barrier()

    for z in range(RPS // ZR):
        r0 = sid * RPS + z * ZR
        pltpu.sync_copy(out_sh.at[pl.ds(r0, ZR)], outp_hbm.at[cid, pl.ds(r0, ZR)])
    pltpu.sync_copy(b2_sh.at[pl.ds(sid * B2S, B2S)],
                    b2p_hbm.at[cid, pl.ds(sid * B2S, B2S)])


_sc_edges = functools.partial(
    pl.kernel,
    out_type=(
        jax.ShapeDtypeStruct((NC, NP, H), jnp.float32),
        jax.ShapeDtypeStruct((NC, B2W), jnp.float32),
    ),
    mesh=plsc.VectorSubcoreMesh(
        core_axis_name="c", subcore_axis_name="s", num_cores=NC,
        num_subcores=NS),
    compiler_params=pltpu.CompilerParams(needs_layout_passes=False),
    scratch_types=[
        pltpu.VMEM((2 * N,), jnp.float32),
        pltpu.VMEM((16,), jnp.float32),
        pltpu.VMEM((K,), jnp.int32),
        pltpu.VMEM((K,), jnp.int32),
        pltpu.VMEM((K,), jnp.int32),
        pltpu.VMEM((K,), jnp.int32),
        pltpu.VMEM((K,), jnp.float32),
        pltpu.VMEM((K, H), jnp.float32),
        pltpu.VMEM((K,), jnp.int32),
        pltpu.VMEM((K,), jnp.int32),
        pltpu.VMEM((K,), jnp.int32),
        pltpu.VMEM((K,), jnp.int32),
        pltpu.VMEM((K,), jnp.int32),
        pltpu.VMEM((K,), jnp.float32),
        pltpu.VMEM((K, H), jnp.float32),
        pltpu.VMEM((K,), jnp.int32),
        pltpu.VMEM_SHARED((NP, H), jnp.float32),
        pltpu.VMEM_SHARED((B2W,), jnp.float32),
        pltpu.SemaphoreType.DMA,
        pltpu.SemaphoreType.DMA,
        pltpu.SemaphoreType.DMA,
        pltpu.SemaphoreType.DMA,
        pltpu.SemaphoreType.DMA,
        pltpu.SemaphoreType.DMA,
    ],
)(_sc_body)


def _tc_norm_body(p_ref, b2_ref, et_ref, o_ref):
    b2 = b2_ref[0] + b2_ref[1]
    s = jnp.sum(b2, axis=-1, keepdims=True)
    et2 = jnp.dot(b2, et_ref[...], preferred_element_type=jnp.float32)
    num = p_ref[0] + p_ref[1] + et2
    o_ref[...] = num * jnp.where(s > 0.0, 1.0 / s, 0.0)


def _tc_norm(p, b2r, etp):
    return pl.pallas_call(
        _tc_norm_body,
        grid=(N // RB,),
        in_specs=[
            pl.BlockSpec((NC, RB, H), lambda i: (0, i, 0)),
            pl.BlockSpec((NC, RB, TP), lambda i: (0, i, 0)),
            pl.BlockSpec((TP, H), lambda i: (0, 0)),
        ],
        out_specs=pl.BlockSpec((RB, H), lambda i: (i, 0)),
        out_shape=jax.ShapeDtypeStruct((N, H), jnp.float32),
    )(p, b2r, etp)


def kernel(inputs, W1, b1, W2, b2, edge_table, source_indices, dest_indices,
           edge_types):
    w2m = jnp.concatenate([W2[:H], W2[H:]], axis=1)
    keys, ab = _tc_head(inputs, W1, b1.reshape(1, H), w2m)
    c6 = jnp.pad(edge_table @ W2[:H, 0] + b2[0], (0, 16 - T))
    z2d = jnp.zeros((ZR, H), jnp.float32)
    z1d = jnp.zeros((B2S,), jnp.float32)
    outp, b2p = _sc_edges(
        keys, ab.reshape(2 * N), c6,
        source_indices.astype(jnp.int32),
        dest_indices.astype(jnp.int32),
        edge_types.astype(jnp.int32),
        z2d, z1d)
    etp = jnp.pad(edge_table, ((0, TP - T), (0, 0)))
    out = _tc_norm(outp, b2p.reshape(NC, NP, TP), etp)
    return out
